# Initial kernel scaffold; baseline (speedup 1.0000x reference)
import jax
import jax.numpy as jnp
from jax.experimental import pallas as pl


def kernel(x, edge_index, batch, W1, b1, W2, b2, Wl, bl):
    raise NotImplementedError("write your pallas kernel here")


# speedup vs baseline: 11.4920x; 11.4920x over previous
import functools

import jax
import jax.numpy as jnp
from jax import lax
from jax.experimental import pallas as pl
from jax.experimental.pallas import tpu as pltpu
from jax.experimental.pallas import tpu_sc as plsc

N = 10000
NPAD = 10240
E = 160000
D = 256
H = 128
G = 128
R = 1024
NBLK = NPAD // R

EC = 80
NCH = E // (16 * EC)
SUB = 25
DC = 40
DCH = E // (32 * DC)
RPT = NPAD // 16

@functools.cache
def _mesh():
    return plsc.VectorSubcoreMesh(
        core_axis_name="c", subcore_axis_name="s", num_cores=2,
        num_subcores=16)


def _deg_body(dst_hbm, degp_hbm, dst_v, ones_v, buf_v, deg_sh):
    cid = lax.axis_index("c")
    sid = lax.axis_index("s")
    t = cid * 16 + sid

    for i in range(RPT // 16):
        buf_v[pl.ds(i * 16, 16)] = jnp.zeros((16,), jnp.float32)
    pltpu.sync_copy(buf_v, deg_sh.at[pl.ds(sid * RPT, RPT)])
    plsc.subcore_barrier()

    for i in range(DC // 16 + 1):
        ones_v[pl.ds(i * 16, 16)] = jnp.ones((16,), jnp.float32)
    pltpu.sync_copy(dst_hbm.at[t], dst_v)

    def chunk(j, _):
        pltpu.sync_copy(ones_v.at[pl.ds(0, DC)], deg_sh.at[dst_v.at[j]],
                        add=True)
        return 0
    lax.fori_loop(0, DCH, chunk, 0)
    plsc.subcore_barrier()

    pltpu.sync_copy(deg_sh.at[pl.ds(sid * RPT, RPT)], buf_v)
    pltpu.sync_copy(buf_v, degp_hbm.at[pl.ds(cid * NPAD + sid * RPT, RPT)])


def _deg_call(dst_slabs):
    flat = pl.kernel(
        _deg_body,
        out_type=jax.ShapeDtypeStruct((2 * NPAD,), jnp.float32),
        mesh=_mesh(),
        scratch_types=[
            pltpu.VMEM((DCH, DC), jnp.int32),
            pltpu.VMEM((48,), jnp.float32),
            pltpu.VMEM((RPT,), jnp.float32),
            pltpu.VMEM_SHARED((NPAD,), jnp.float32),
        ],
    )(dst_slabs)
    return jnp.stack([flat[:NPAD], flat[NPAD:]], axis=1)


def _prop_body(g_hbm, src_hbm, dst_hbm, s_hbm,
               src_v, dst_v, rows_v, accum, sem):
    cid = lax.axis_index("c")
    sid = lax.axis_index("s")

    def zrow(r, _):
        for b in range(H // 16):
            rows_v[r, pl.ds(b * 16, 16)] = jnp.zeros((16,), jnp.float32)
        return 0
    lax.fori_loop(0, EC, zrow, 0)
    for k in range(RPT // EC):
        pltpu.sync_copy(rows_v, accum.at[pl.ds(sid * RPT + k * EC, EC), :])
    plsc.subcore_barrier()

    def outer(o, _):
        pltpu.sync_copy(src_hbm.at[cid, sid, o], src_v)
        pltpu.sync_copy(dst_hbm.at[sid, o], dst_v)

        def chunk(j, _):
            pltpu.async_copy(g_hbm.at[src_v.at[j]], rows_v, sem).wait()
            pltpu.sync_copy(rows_v, accum.at[dst_v.at[j]], add=True)
            return 0
        lax.fori_loop(0, SUB, chunk, 0)
        return 0
    lax.fori_loop(0, NCH // SUB, outer, 0)
    plsc.subcore_barrier()

    pltpu.sync_copy(accum.at[pl.ds(sid * RPT, RPT), :],
                    s_hbm.at[pl.ds(cid * NPAD + sid * RPT, RPT), :])


def _prop_call(g_flat, src_slabs, dst_slabs):
    return pl.kernel(
        _prop_body,
        out_type=jax.ShapeDtypeStruct((2 * NPAD, H), jnp.float32),
        mesh=_mesh(),
        scratch_types=[
            pltpu.VMEM((SUB, EC), jnp.int32),
            pltpu.VMEM((SUB, EC), jnp.int32),
            pltpu.VMEM((EC, H), jnp.float32),
            pltpu.VMEM_SHARED((NPAD, H), jnp.float32),
            pltpu.SemaphoreType.DMA,
        ],
    )(g_flat, src_slabs, dst_slabs)


def _dinv(dp_ref):
    deg = dp_ref[:, 0] + dp_ref[:, 1] + 1.0
    return lax.rsqrt(deg)


def _tc1_body(x_ref, w_ref, dp_ref, o_ref):
    dinv = _dinv(dp_ref)
    h = jnp.dot(x_ref[...], w_ref[...], preferred_element_type=jnp.float32)
    o_ref[...] = h * dinv[:, None]


def _tc1_call(xp, W1, degp):
    return pl.pallas_call(
        _tc1_body,
        grid=(2, NBLK),
        in_specs=[
            pl.BlockSpec((R, D), lambda c, r: (r, 0)),
            pl.BlockSpec((D, H), lambda c, r: (0, c)),
            pl.BlockSpec((R, 2), lambda c, r: (r, 0)),
        ],
        out_specs=pl.BlockSpec((R, H), lambda c, r: (c * NBLK + r, 0)),
        out_shape=jax.ShapeDtypeStruct((2 * NPAD, H), jnp.float32),
    )(xp, W1, degp)


def _tc2_body(sa_ref, sb_ref, ga_ref, gb_ref, dp_ref, b_ref, w_ref, o_ref):
    dinv = _dinv(dp_ref)[:, None]
    ha = jnp.maximum((sa_ref[...] + ga_ref[...]) * dinv + b_ref[0, :H], 0.0)
    hb = jnp.maximum((sb_ref[...] + gb_ref[...]) * dinv + b_ref[0, H:], 0.0)
    h2 = (jnp.dot(ha, w_ref[:H, :], preferred_element_type=jnp.float32)
          + jnp.dot(hb, w_ref[H:, :], preferred_element_type=jnp.float32))
    o_ref[...] = h2 * dinv


def _tc2_call(s_flat, g_flat, degp, b1, W2):
    half_a = pl.BlockSpec((R, H), lambda c, r: (r, 0))
    half_b = pl.BlockSpec((R, H), lambda c, r: (NBLK + r, 0))
    return pl.pallas_call(
        _tc2_body,
        grid=(2, NBLK),
        in_specs=[
            half_a, half_b, half_a, half_b,
            pl.BlockSpec((R, 2), lambda c, r: (r, 0)),
            pl.BlockSpec((1, D), lambda c, r: (0, 0)),
            pl.BlockSpec((D, H), lambda c, r: (0, c)),
        ],
        out_specs=pl.BlockSpec((R, H), lambda c, r: (c * NBLK + r, 0)),
        out_shape=jax.ShapeDtypeStruct((2 * NPAD, H), jnp.float32),
    )(s_flat, s_flat, g_flat, g_flat, degp, b1, W2)


def _tc3_body(sa_ref, sb_ref, ga_ref, gb_ref, dp_ref, b_ref, wl_ref, bl_ref,
              bat_ref, o_ref):
    r = pl.program_id(0)
    dinv = _dinv(dp_ref)[:, None]
    ha = jnp.maximum((sa_ref[...] + ga_ref[...]) * dinv + b_ref[0, :H], 0.0)
    hb = jnp.maximum((sb_ref[...] + gb_ref[...]) * dinv + b_ref[0, H:], 0.0)
    y = (jnp.dot(ha, wl_ref[:H, :], preferred_element_type=jnp.float32)
         + jnp.dot(hb, wl_ref[H:, :], preferred_element_type=jnp.float32))
    gids = lax.broadcasted_iota(jnp.int32, (R, G), 1)
    p = (bat_ref[...] == gids).astype(jnp.float32)
    contrib = lax.dot_general(p, y, (((0,), (0,)), ((), ())),
                              preferred_element_type=jnp.float32)

    @pl.when(r == 0)
    def _():
        o_ref[...] = contrib + bl_ref[...]

    @pl.when(r != 0)
    def _():
        o_ref[...] = o_ref[...] + contrib


def _tc3_call(s_flat, g_flat, degp, b2, Wl, bl, batchp):
    half_a = pl.BlockSpec((R, H), lambda r: (r, 0))
    half_b = pl.BlockSpec((R, H), lambda r: (NBLK + r, 0))
    return pl.pallas_call(
        _tc3_body,
        grid=(NBLK,),
        in_specs=[
            half_a, half_b, half_a, half_b,
            pl.BlockSpec((R, 2), lambda r: (r, 0)),
            pl.BlockSpec((1, D), lambda r: (0, 0)),
            pl.BlockSpec((D, 64), lambda r: (0, 0)),
            pl.BlockSpec((1, 64), lambda r: (0, 0)),
            pl.BlockSpec((R, 1), lambda r: (r, 0)),
        ],
        out_specs=pl.BlockSpec((G, 64), lambda r: (0, 0)),
        out_shape=jax.ShapeDtypeStruct((G, 64), jnp.float32),
    )(s_flat, s_flat, g_flat, g_flat, degp, b2, Wl, bl, batchp)


def kernel(x, edge_index, batch, W1, b1, W2, b2, Wl, bl):
    src = edge_index[0]
    dst = edge_index[1]

    xp = jnp.pad(x, ((0, NPAD - N), (0, 0)))
    batchp = jnp.concatenate(
        [batch, jnp.full((NPAD - N,), G, dtype=batch.dtype)]).reshape(NPAD, 1)
    src_t = src.reshape(16, NCH // SUB, SUB, EC)
    src_slabs = jnp.stack([src_t, src_t + NPAD])
    dst_slabs = dst.reshape(16, NCH // SUB, SUB, EC)
    dst_deg = dst.reshape(32, DCH, DC)
    b1r = b1.reshape(1, D)
    b2r = b2.reshape(1, D)
    blr = bl.reshape(1, 64)

    degp = _deg_call(dst_deg)
    g1 = _tc1_call(xp, W1, degp)
    s1 = _prop_call(g1, src_slabs, dst_slabs)
    g2 = _tc2_call(s1, g1, degp, b1r, W2)
    s2 = _prop_call(g2, src_slabs, dst_slabs)
    return _tc3_call(s2, g2, degp, b2r, Wl, blr, batchp)


# speedup vs baseline: 14.1424x; 1.2306x over previous
import functools

import jax
import jax.numpy as jnp
from jax import lax
from jax.experimental import pallas as pl
from jax.experimental.pallas import tpu as pltpu
from jax.experimental.pallas import tpu_sc as plsc

N = 10000
NPAD = 10240
E = 160000
D = 256
H = 128
G = 128
R = 1024
NBLK = NPAD // R

EC = 80
NCH = E // (16 * EC)
SUB = 25
DC = 40
DCH = E // (32 * DC)
RPT = NPAD // 16

@functools.cache
def _mesh():
    return plsc.VectorSubcoreMesh(
        core_axis_name="c", subcore_axis_name="s", num_cores=2,
        num_subcores=16)


def _deg_body(dst_hbm, degp_hbm, dst_v, ones_v, buf_v, deg_sh):
    cid = lax.axis_index("c")
    sid = lax.axis_index("s")
    t = cid * 16 + sid

    for i in range(RPT // 16):
        buf_v[pl.ds(i * 16, 16)] = jnp.zeros((16,), jnp.float32)
    pltpu.sync_copy(buf_v, deg_sh.at[pl.ds(sid * RPT, RPT)])
    plsc.subcore_barrier()

    for i in range(DC // 16 + 1):
        ones_v[pl.ds(i * 16, 16)] = jnp.ones((16,), jnp.float32)
    pltpu.sync_copy(dst_hbm.at[t], dst_v)

    def chunk(j, _):
        pltpu.sync_copy(ones_v.at[pl.ds(0, DC)], deg_sh.at[dst_v.at[j]],
                        add=True)
        return 0
    lax.fori_loop(0, DCH, chunk, 0)
    plsc.subcore_barrier()

    pltpu.sync_copy(deg_sh.at[pl.ds(sid * RPT, RPT)], buf_v)
    pltpu.sync_copy(buf_v, degp_hbm.at[pl.ds(cid * NPAD + sid * RPT, RPT)])


def _deg_call(dst_slabs):
    flat = pl.kernel(
        _deg_body,
        out_type=jax.ShapeDtypeStruct((2 * NPAD,), jnp.float32),
        mesh=_mesh(),
        scratch_types=[
            pltpu.VMEM((DCH, DC), jnp.int32),
            pltpu.VMEM((48,), jnp.float32),
            pltpu.VMEM((RPT,), jnp.float32),
            pltpu.VMEM_SHARED((NPAD,), jnp.float32),
        ],
    )(dst_slabs)
    return jnp.stack([flat[:NPAD], flat[NPAD:]], axis=1)


def _prop_body(g_hbm, src_hbm, dst_hbm, s_hbm,
               src_v, dst_v, rows0_v, rows1_v, accum, sem0, sem1):
    cid = lax.axis_index("c")
    sid = lax.axis_index("s")

    def zrow(r, _):
        for b in range(H // 16):
            rows0_v[r, pl.ds(b * 16, 16)] = jnp.zeros((16,), jnp.float32)
        return 0
    lax.fori_loop(0, EC, zrow, 0)
    for k in range(RPT // EC):
        pltpu.sync_copy(rows0_v, accum.at[pl.ds(sid * RPT + k * EC, EC), :])
    plsc.subcore_barrier()

    def outer(o, _):
        pltpu.sync_copy(src_hbm.at[cid, sid, o], src_v)
        pltpu.sync_copy(dst_hbm.at[sid, o], dst_v)
        pltpu.async_copy(g_hbm.at[src_v.at[0]], rows0_v, sem0)

        def chunk(j, _):
            even = (j & 1) == 0

            @pl.when(even)
            def _():
                pltpu.make_async_copy(
                    g_hbm.at[src_v.at[j]], rows0_v, sem0).wait()

                @pl.when(j < SUB - 1)
                def _():
                    pltpu.async_copy(g_hbm.at[src_v.at[j + 1]], rows1_v, sem1)
                pltpu.sync_copy(rows0_v, accum.at[dst_v.at[j]], add=True)

            @pl.when(jnp.logical_not(even))
            def _():
                pltpu.make_async_copy(
                    g_hbm.at[src_v.at[j]], rows1_v, sem1).wait()

                @pl.when(j < SUB - 1)
                def _():
                    pltpu.async_copy(g_hbm.at[src_v.at[j + 1]], rows0_v, sem0)
                pltpu.sync_copy(rows1_v, accum.at[dst_v.at[j]], add=True)
            return 0
        lax.fori_loop(0, SUB, chunk, 0)
        return 0
    lax.fori_loop(0, NCH // SUB, outer, 0)
    plsc.subcore_barrier()

    pltpu.sync_copy(accum.at[pl.ds(sid * RPT, RPT), :],
                    s_hbm.at[pl.ds(cid * NPAD + sid * RPT, RPT), :])


def _prop_call(g_flat, src_slabs, dst_slabs):
    return pl.kernel(
        _prop_body,
        out_type=jax.ShapeDtypeStruct((2 * NPAD, H), jnp.float32),
        mesh=_mesh(),
        scratch_types=[
            pltpu.VMEM((SUB, EC), jnp.int32),
            pltpu.VMEM((SUB, EC), jnp.int32),
            pltpu.VMEM((EC, H), jnp.float32),
            pltpu.VMEM((EC, H), jnp.float32),
            pltpu.VMEM_SHARED((NPAD, H), jnp.float32),
            pltpu.SemaphoreType.DMA,
            pltpu.SemaphoreType.DMA,
        ],
    )(g_flat, src_slabs, dst_slabs)


def _dinv(dp_ref):
    deg = dp_ref[:, 0] + dp_ref[:, 1] + 1.0
    return lax.rsqrt(deg)


def _tc1_body(x_ref, w_ref, dp_ref, o_ref):
    dinv = _dinv(dp_ref)
    h = jnp.dot(x_ref[...], w_ref[...], preferred_element_type=jnp.float32)
    o_ref[...] = h * dinv[:, None]


def _tc1_call(xp, W1, degp):
    return pl.pallas_call(
        _tc1_body,
        grid=(2, NBLK),
        in_specs=[
            pl.BlockSpec((R, D), lambda c, r: (r, 0)),
            pl.BlockSpec((D, H), lambda c, r: (0, c)),
            pl.BlockSpec((R, 2), lambda c, r: (r, 0)),
        ],
        out_specs=pl.BlockSpec((R, H), lambda c, r: (c * NBLK + r, 0)),
        out_shape=jax.ShapeDtypeStruct((2 * NPAD, H), jnp.float32),
    )(xp, W1, degp)


def _tc2_body(sa_ref, sb_ref, ga_ref, gb_ref, dp_ref, b_ref, w_ref, o_ref):
    dinv = _dinv(dp_ref)[:, None]
    ha = jnp.maximum((sa_ref[...] + ga_ref[...]) * dinv + b_ref[0, :H], 0.0)
    hb = jnp.maximum((sb_ref[...] + gb_ref[...]) * dinv + b_ref[0, H:], 0.0)
    h2 = (jnp.dot(ha, w_ref[:H, :], preferred_element_type=jnp.float32)
          + jnp.dot(hb, w_ref[H:, :], preferred_element_type=jnp.float32))
    o_ref[...] = h2 * dinv


def _tc2_call(s_flat, g_flat, degp, b1, W2):
    half_a = pl.BlockSpec((R, H), lambda c, r: (r, 0))
    half_b = pl.BlockSpec((R, H), lambda c, r: (NBLK + r, 0))
    return pl.pallas_call(
        _tc2_body,
        grid=(2, NBLK),
        in_specs=[
            half_a, half_b, half_a, half_b,
            pl.BlockSpec((R, 2), lambda c, r: (r, 0)),
            pl.BlockSpec((1, D), lambda c, r: (0, 0)),
            pl.BlockSpec((D, H), lambda c, r: (0, c)),
        ],
        out_specs=pl.BlockSpec((R, H), lambda c, r: (c * NBLK + r, 0)),
        out_shape=jax.ShapeDtypeStruct((2 * NPAD, H), jnp.float32),
    )(s_flat, s_flat, g_flat, g_flat, degp, b1, W2)


def _tc3_body(sa_ref, sb_ref, ga_ref, gb_ref, dp_ref, b_ref, wl_ref, bl_ref,
              bat_ref, o_ref):
    r = pl.program_id(0)
    dinv = _dinv(dp_ref)[:, None]
    ha = jnp.maximum((sa_ref[...] + ga_ref[...]) * dinv + b_ref[0, :H], 0.0)
    hb = jnp.maximum((sb_ref[...] + gb_ref[...]) * dinv + b_ref[0, H:], 0.0)
    y = (jnp.dot(ha, wl_ref[:H, :], preferred_element_type=jnp.float32)
         + jnp.dot(hb, wl_ref[H:, :], preferred_element_type=jnp.float32))
    gids = lax.broadcasted_iota(jnp.int32, (R, G), 1)
    p = (bat_ref[...] == gids).astype(jnp.float32)
    contrib = lax.dot_general(p, y, (((0,), (0,)), ((), ())),
                              preferred_element_type=jnp.float32)

    @pl.when(r == 0)
    def _():
        o_ref[...] = contrib + bl_ref[...]

    @pl.when(r != 0)
    def _():
        o_ref[...] = o_ref[...] + contrib


def _tc3_call(s_flat, g_flat, degp, b2, Wl, bl, batchp):
    half_a = pl.BlockSpec((R, H), lambda r: (r, 0))
    half_b = pl.BlockSpec((R, H), lambda r: (NBLK + r, 0))
    return pl.pallas_call(
        _tc3_body,
        grid=(NBLK,),
        in_specs=[
            half_a, half_b, half_a, half_b,
            pl.BlockSpec((R, 2), lambda r: (r, 0)),
            pl.BlockSpec((1, D), lambda r: (0, 0)),
            pl.BlockSpec((D, 64), lambda r: (0, 0)),
            pl.BlockSpec((1, 64), lambda r: (0, 0)),
            pl.BlockSpec((R, 1), lambda r: (r, 0)),
        ],
        out_specs=pl.BlockSpec((G, 64), lambda r: (0, 0)),
        out_shape=jax.ShapeDtypeStruct((G, 64), jnp.float32),
    )(s_flat, s_flat, g_flat, g_flat, degp, b2, Wl, bl, batchp)


def kernel(x, edge_index, batch, W1, b1, W2, b2, Wl, bl):
    src = edge_index[0]
    dst = edge_index[1]

    xp = jnp.pad(x, ((0, NPAD - N), (0, 0)))
    batchp = jnp.concatenate(
        [batch, jnp.full((NPAD - N,), G, dtype=batch.dtype)]).reshape(NPAD, 1)
    src_t = src.reshape(16, NCH // SUB, SUB, EC)
    src_slabs = jnp.stack([src_t, src_t + NPAD])
    dst_slabs = dst.reshape(16, NCH // SUB, SUB, EC)
    dst_deg = dst.reshape(32, DCH, DC)
    b1r = b1.reshape(1, D)
    b2r = b2.reshape(1, D)
    blr = bl.reshape(1, 64)

    degp = _deg_call(dst_deg)
    g1 = _tc1_call(xp, W1, degp)
    s1 = _prop_call(g1, src_slabs, dst_slabs)
    g2 = _tc2_call(s1, g1, degp, b1r, W2)
    s2 = _prop_call(g2, src_slabs, dst_slabs)
    return _tc3_call(s2, g2, degp, b2r, Wl, blr, batchp)


# speedup vs baseline: 18.8565x; 1.3333x over previous
import functools

import jax
import jax.numpy as jnp
from jax import lax
from jax.experimental import pallas as pl
from jax.experimental.pallas import tpu as pltpu
from jax.experimental.pallas import tpu_sc as plsc

N = 10000
NPAD = 10240
E = 160000
D = 256
H = 128
G = 128
R = 1024
NBLK = NPAD // R

EC = 80
NCH = E // (16 * EC)
SUB = 25
DC = 40
DCH = E // (32 * DC)
RPT = NPAD // 16

@functools.cache
def _mesh():
    return plsc.VectorSubcoreMesh(
        core_axis_name="c", subcore_axis_name="s", num_cores=2,
        num_subcores=16)


def _deg_body(dst_hbm, degp_hbm, dst_v, ones_v, buf_v, deg_sh):
    cid = lax.axis_index("c")
    sid = lax.axis_index("s")
    t = cid * 16 + sid

    for i in range(RPT // 16):
        buf_v[pl.ds(i * 16, 16)] = jnp.zeros((16,), jnp.float32)
    pltpu.sync_copy(buf_v, deg_sh.at[pl.ds(sid * RPT, RPT)])
    plsc.subcore_barrier()

    for i in range(DC // 16 + 1):
        ones_v[pl.ds(i * 16, 16)] = jnp.ones((16,), jnp.float32)
    pltpu.sync_copy(dst_hbm.at[t], dst_v)

    def chunk(j, _):
        pltpu.sync_copy(ones_v.at[pl.ds(0, DC)], deg_sh.at[dst_v.at[j]],
                        add=True)
        return 0
    lax.fori_loop(0, DCH, chunk, 0)
    plsc.subcore_barrier()

    pltpu.sync_copy(deg_sh.at[pl.ds(sid * RPT, RPT)], buf_v)
    pltpu.sync_copy(buf_v, degp_hbm.at[pl.ds(cid * NPAD + sid * RPT, RPT)])


def _deg_call(dst_slabs):
    flat = pl.kernel(
        _deg_body,
        out_type=jax.ShapeDtypeStruct((2 * NPAD,), jnp.float32),
        mesh=_mesh(),
        scratch_types=[
            pltpu.VMEM((DCH, DC), jnp.int32),
            pltpu.VMEM((48,), jnp.float32),
            pltpu.VMEM((RPT,), jnp.float32),
            pltpu.VMEM_SHARED((NPAD,), jnp.float32),
        ],
    )(dst_slabs)
    return jnp.stack([flat[:NPAD], flat[NPAD:]], axis=1)


def _prop_body(g_hbm, src_hbm, dst_hbm, s_hbm,
               src_v, dst_v, rows0_v, rows1_v, rows2_v, accum,
               sem0, sem1, sem2):
    cid = lax.axis_index("c")
    sid = lax.axis_index("s")

    def zrow(r, _):
        for b in range(H // 16):
            rows0_v[r, pl.ds(b * 16, 16)] = jnp.zeros((16,), jnp.float32)
        return 0
    lax.fori_loop(0, EC, zrow, 0)
    for k in range(RPT // EC):
        pltpu.sync_copy(rows0_v, accum.at[pl.ds(sid * RPT + k * EC, EC), :])
    plsc.subcore_barrier()

    rows = (rows0_v, rows1_v, rows2_v)
    sems = (sem0, sem1, sem2)

    def outer(o, _):
        pltpu.sync_copy(src_hbm.at[cid, sid, o], src_v)
        pltpu.sync_copy(dst_hbm.at[sid, o], dst_v)
        pltpu.async_copy(g_hbm.at[src_v.at[0]], rows[0], sems[0])
        pltpu.async_copy(g_hbm.at[src_v.at[1]], rows[1], sems[1])

        def chunk(j, _):
            for b in range(3):
                @pl.when(j % 3 == b)
                def _():
                    pltpu.make_async_copy(
                        g_hbm.at[src_v.at[j]], rows[b], sems[b]).wait()

                    @pl.when(j < SUB - 2)
                    def _():
                        pltpu.async_copy(
                            g_hbm.at[src_v.at[j + 2]], rows[(b + 2) % 3],
                            sems[(b + 2) % 3])
                    pltpu.sync_copy(rows[b], accum.at[dst_v.at[j]], add=True)
            return 0
        lax.fori_loop(0, SUB, chunk, 0)
        return 0
    lax.fori_loop(0, NCH // SUB, outer, 0)
    plsc.subcore_barrier()

    pltpu.sync_copy(accum.at[pl.ds(sid * RPT, RPT), :],
                    s_hbm.at[pl.ds(cid * NPAD + sid * RPT, RPT), :])


def _prop_call(g_flat, src_slabs, dst_slabs):
    return pl.kernel(
        _prop_body,
        out_type=jax.ShapeDtypeStruct((2 * NPAD, H), jnp.float32),
        mesh=_mesh(),
        scratch_types=[
            pltpu.VMEM((SUB, EC), jnp.int32),
            pltpu.VMEM((SUB, EC), jnp.int32),
            pltpu.VMEM((EC, H), jnp.float32),
            pltpu.VMEM((EC, H), jnp.float32),
            pltpu.VMEM((EC, H), jnp.float32),
            pltpu.VMEM_SHARED((NPAD, H), jnp.float32),
            pltpu.SemaphoreType.DMA,
            pltpu.SemaphoreType.DMA,
            pltpu.SemaphoreType.DMA,
        ],
    )(g_flat, src_slabs, dst_slabs)


def _dinv(dp_ref):
    deg = dp_ref[:, 0] + dp_ref[:, 1] + 1.0
    return lax.rsqrt(deg)


def _tc1_body(x_ref, w_ref, dp_ref, o_ref):
    dinv = _dinv(dp_ref)
    h = jnp.dot(x_ref[...], w_ref[...], preferred_element_type=jnp.float32)
    o_ref[...] = h * dinv[:, None]


def _tc1_call(xp, W1, degp):
    return pl.pallas_call(
        _tc1_body,
        grid=(2, NBLK),
        in_specs=[
            pl.BlockSpec((R, D), lambda c, r: (r, 0)),
            pl.BlockSpec((D, H), lambda c, r: (0, c)),
            pl.BlockSpec((R, 2), lambda c, r: (r, 0)),
        ],
        out_specs=pl.BlockSpec((R, H), lambda c, r: (c * NBLK + r, 0)),
        out_shape=jax.ShapeDtypeStruct((2 * NPAD, H), jnp.float32),
    )(xp, W1, degp)


def _tc2_body(sa_ref, sb_ref, ga_ref, gb_ref, dp_ref, b_ref, w_ref, o_ref):
    dinv = _dinv(dp_ref)[:, None]
    ha = jnp.maximum((sa_ref[...] + ga_ref[...]) * dinv + b_ref[0, :H], 0.0)
    hb = jnp.maximum((sb_ref[...] + gb_ref[...]) * dinv + b_ref[0, H:], 0.0)
    h2 = (jnp.dot(ha, w_ref[:H, :], preferred_element_type=jnp.float32)
          + jnp.dot(hb, w_ref[H:, :], preferred_element_type=jnp.float32))
    o_ref[...] = h2 * dinv


def _tc2_call(s_flat, g_flat, degp, b1, W2):
    half_a = pl.BlockSpec((R, H), lambda c, r: (r, 0))
    half_b = pl.BlockSpec((R, H), lambda c, r: (NBLK + r, 0))
    return pl.pallas_call(
        _tc2_body,
        grid=(2, NBLK),
        in_specs=[
            half_a, half_b, half_a, half_b,
            pl.BlockSpec((R, 2), lambda c, r: (r, 0)),
            pl.BlockSpec((1, D), lambda c, r: (0, 0)),
            pl.BlockSpec((D, H), lambda c, r: (0, c)),
        ],
        out_specs=pl.BlockSpec((R, H), lambda c, r: (c * NBLK + r, 0)),
        out_shape=jax.ShapeDtypeStruct((2 * NPAD, H), jnp.float32),
    )(s_flat, s_flat, g_flat, g_flat, degp, b1, W2)


def _tc3_body(sa_ref, sb_ref, ga_ref, gb_ref, dp_ref, b_ref, wl_ref, bl_ref,
              bat_ref, o_ref):
    r = pl.program_id(0)
    dinv = _dinv(dp_ref)[:, None]
    ha = jnp.maximum((sa_ref[...] + ga_ref[...]) * dinv + b_ref[0, :H], 0.0)
    hb = jnp.maximum((sb_ref[...] + gb_ref[...]) * dinv + b_ref[0, H:], 0.0)
    y = (jnp.dot(ha, wl_ref[:H, :], preferred_element_type=jnp.float32)
         + jnp.dot(hb, wl_ref[H:, :], preferred_element_type=jnp.float32))
    gids = lax.broadcasted_iota(jnp.int32, (R, G), 1)
    p = (bat_ref[...] == gids).astype(jnp.float32)
    contrib = lax.dot_general(p, y, (((0,), (0,)), ((), ())),
                              preferred_element_type=jnp.float32)

    @pl.when(r == 0)
    def _():
        o_ref[...] = contrib + bl_ref[...]

    @pl.when(r != 0)
    def _():
        o_ref[...] = o_ref[...] + contrib


def _tc3_call(s_flat, g_flat, degp, b2, Wl, bl, batchp):
    half_a = pl.BlockSpec((R, H), lambda r: (r, 0))
    half_b = pl.BlockSpec((R, H), lambda r: (NBLK + r, 0))
    return pl.pallas_call(
        _tc3_body,
        grid=(NBLK,),
        in_specs=[
            half_a, half_b, half_a, half_b,
            pl.BlockSpec((R, 2), lambda r: (r, 0)),
            pl.BlockSpec((1, D), lambda r: (0, 0)),
            pl.BlockSpec((D, 64), lambda r: (0, 0)),
            pl.BlockSpec((1, 64), lambda r: (0, 0)),
            pl.BlockSpec((R, 1), lambda r: (r, 0)),
        ],
        out_specs=pl.BlockSpec((G, 64), lambda r: (0, 0)),
        out_shape=jax.ShapeDtypeStruct((G, 64), jnp.float32),
    )(s_flat, s_flat, g_flat, g_flat, degp, b2, Wl, bl, batchp)


def kernel(x, edge_index, batch, W1, b1, W2, b2, Wl, bl):
    src = edge_index[0]
    dst = edge_index[1]

    xp = jnp.pad(x, ((0, NPAD - N), (0, 0)))
    batchp = jnp.concatenate(
        [batch, jnp.full((NPAD - N,), G, dtype=batch.dtype)]).reshape(NPAD, 1)
    src_t = src.reshape(16, NCH // SUB, SUB, EC)
    src_slabs = jnp.stack([src_t, src_t + NPAD])
    dst_slabs = dst.reshape(16, NCH // SUB, SUB, EC)
    dst_deg = dst.reshape(32, DCH, DC)
    b1r = b1.reshape(1, D)
    b2r = b2.reshape(1, D)
    blr = bl.reshape(1, 64)

    degp = _deg_call(dst_deg)
    g1 = _tc1_call(xp, W1, degp)
    s1 = _prop_call(g1, src_slabs, dst_slabs)
    g2 = _tc2_call(s1, g1, degp, b1r, W2)
    s2 = _prop_call(g2, src_slabs, dst_slabs)
    return _tc3_call(s2, g2, degp, b2r, Wl, blr, batchp)


# speedup vs baseline: 18.9865x; 1.0069x over previous
import functools

import jax
import jax.numpy as jnp
from jax import lax
from jax.experimental import pallas as pl
from jax.experimental.pallas import tpu as pltpu
from jax.experimental.pallas import tpu_sc as plsc

N = 10000
NPAD = 10240
E = 160000
D = 256
H = 128
G = 128
R = 1024
NBLK = NPAD // R

EC = 80
NCH = E // (16 * EC)
SUB = 25
DC = 40
DCH = E // (32 * DC)
RPT = NPAD // 16

@functools.cache
def _mesh():
    return plsc.VectorSubcoreMesh(
        core_axis_name="c", subcore_axis_name="s", num_cores=2,
        num_subcores=16)


def _deg_body(dst_hbm, degp_hbm, dst_v, ones_v, buf_v, deg_sh):
    cid = lax.axis_index("c")
    sid = lax.axis_index("s")
    t = cid * 16 + sid

    for i in range(RPT // 16):
        buf_v[pl.ds(i * 16, 16)] = jnp.zeros((16,), jnp.float32)
    pltpu.sync_copy(buf_v, deg_sh.at[pl.ds(sid * RPT, RPT)])
    plsc.subcore_barrier()

    for i in range(DC // 16 + 1):
        ones_v[pl.ds(i * 16, 16)] = jnp.ones((16,), jnp.float32)

    def outer(o, _):
        pltpu.sync_copy(dst_hbm.at[t, o], dst_v)

        def chunk(j, _):
            pltpu.sync_copy(ones_v.at[pl.ds(0, DC)], deg_sh.at[dst_v.at[j]],
                            add=True)
            return 0
        lax.fori_loop(0, DCH // 5, chunk, 0)
        return 0
    lax.fori_loop(0, 5, outer, 0)
    plsc.subcore_barrier()

    pltpu.sync_copy(deg_sh.at[pl.ds(sid * RPT, RPT)], buf_v)
    pltpu.sync_copy(buf_v, degp_hbm.at[pl.ds(cid * NPAD + sid * RPT, RPT)])


def _deg_call(dst_slabs):
    flat = pl.kernel(
        _deg_body,
        out_type=jax.ShapeDtypeStruct((2 * NPAD,), jnp.float32),
        mesh=_mesh(),
        scratch_types=[
            pltpu.VMEM((DCH // 5, DC), jnp.int32),
            pltpu.VMEM((48,), jnp.float32),
            pltpu.VMEM((RPT,), jnp.float32),
            pltpu.VMEM_SHARED((NPAD,), jnp.float32),
        ],
    )(dst_slabs)
    return jnp.stack([flat[:NPAD], flat[NPAD:]], axis=1)


def _prop_body(g_hbm, src_hbm, dst_hbm, s_hbm,
               src_v, dst_v, rows0_v, rows1_v, rows2_v, rows3_v, accum,
               sem0, sem1, sem2, sem3):
    cid = lax.axis_index("c")
    sid = lax.axis_index("s")

    def zrow(r, _):
        for b in range(H // 16):
            rows0_v[r, pl.ds(b * 16, 16)] = jnp.zeros((16,), jnp.float32)
        return 0
    lax.fori_loop(0, EC, zrow, 0)
    for k in range(RPT // EC):
        pltpu.sync_copy(rows0_v, accum.at[pl.ds(sid * RPT + k * EC, EC), :])
    plsc.subcore_barrier()

    rows = (rows0_v, rows1_v, rows2_v, rows3_v)
    sems = (sem0, sem1, sem2, sem3)
    nb = len(rows)

    def outer(o, _):
        pltpu.sync_copy(src_hbm.at[cid, sid, o], src_v)
        pltpu.sync_copy(dst_hbm.at[sid, o], dst_v)
        for b in range(nb - 1):
            pltpu.async_copy(g_hbm.at[src_v.at[b]], rows[b], sems[b])

        def chunk(j, _):
            for b in range(nb):
                @pl.when(j % nb == b)
                def _():
                    pltpu.make_async_copy(
                        g_hbm.at[src_v.at[j]], rows[b], sems[b]).wait()

                    @pl.when(j < SUB - (nb - 1))
                    def _():
                        pltpu.async_copy(
                            g_hbm.at[src_v.at[j + nb - 1]], rows[(b - 1) % nb],
                            sems[(b - 1) % nb])
                    pltpu.sync_copy(rows[b], accum.at[dst_v.at[j]], add=True)
            return 0
        lax.fori_loop(0, SUB, chunk, 0)
        return 0
    lax.fori_loop(0, NCH // SUB, outer, 0)
    plsc.subcore_barrier()

    pltpu.sync_copy(accum.at[pl.ds(sid * RPT, RPT), :],
                    s_hbm.at[pl.ds(cid * NPAD + sid * RPT, RPT), :])


def _prop_call(g_flat, src_slabs, dst_slabs):
    return pl.kernel(
        _prop_body,
        out_type=jax.ShapeDtypeStruct((2 * NPAD, H), jnp.float32),
        mesh=_mesh(),
        scratch_types=[
            pltpu.VMEM((SUB, EC), jnp.int32),
            pltpu.VMEM((SUB, EC), jnp.int32),
            pltpu.VMEM((EC, H), jnp.float32),
            pltpu.VMEM((EC, H), jnp.float32),
            pltpu.VMEM((EC, H), jnp.float32),
            pltpu.VMEM((EC, H), jnp.float32),
            pltpu.VMEM_SHARED((NPAD, H), jnp.float32),
            pltpu.SemaphoreType.DMA,
            pltpu.SemaphoreType.DMA,
            pltpu.SemaphoreType.DMA,
            pltpu.SemaphoreType.DMA,
        ],
    )(g_flat, src_slabs, dst_slabs)


def _dinv(dp_ref):
    deg = dp_ref[:, 0] + dp_ref[:, 1] + 1.0
    return lax.rsqrt(deg)


def _tc1_body(x_ref, w_ref, dp_ref, o_ref):
    dinv = _dinv(dp_ref)
    h = jnp.dot(x_ref[...], w_ref[...], preferred_element_type=jnp.float32)
    o_ref[...] = h * dinv[:, None]


def _tc1_call(xp, W1, degp):
    return pl.pallas_call(
        _tc1_body,
        grid=(2, NBLK),
        in_specs=[
            pl.BlockSpec((R, D), lambda c, r: (r, 0)),
            pl.BlockSpec((D, H), lambda c, r: (0, c)),
            pl.BlockSpec((R, 2), lambda c, r: (r, 0)),
        ],
        out_specs=pl.BlockSpec((R, H), lambda c, r: (c * NBLK + r, 0)),
        out_shape=jax.ShapeDtypeStruct((2 * NPAD, H), jnp.float32),
    )(xp, W1, degp)


def _tc2_body(sa_ref, sb_ref, ga_ref, gb_ref, dp_ref, b_ref, w_ref, o_ref):
    dinv = _dinv(dp_ref)[:, None]
    ha = jnp.maximum((sa_ref[...] + ga_ref[...]) * dinv + b_ref[0, :H], 0.0)
    hb = jnp.maximum((sb_ref[...] + gb_ref[...]) * dinv + b_ref[0, H:], 0.0)
    h2 = (jnp.dot(ha, w_ref[:H, :], preferred_element_type=jnp.float32)
          + jnp.dot(hb, w_ref[H:, :], preferred_element_type=jnp.float32))
    o_ref[...] = h2 * dinv


def _tc2_call(s_flat, g_flat, degp, b1, W2):
    half_a = pl.BlockSpec((R, H), lambda c, r: (r, 0))
    half_b = pl.BlockSpec((R, H), lambda c, r: (NBLK + r, 0))
    return pl.pallas_call(
        _tc2_body,
        grid=(2, NBLK),
        in_specs=[
            half_a, half_b, half_a, half_b,
            pl.BlockSpec((R, 2), lambda c, r: (r, 0)),
            pl.BlockSpec((1, D), lambda c, r: (0, 0)),
            pl.BlockSpec((D, H), lambda c, r: (0, c)),
        ],
        out_specs=pl.BlockSpec((R, H), lambda c, r: (c * NBLK + r, 0)),
        out_shape=jax.ShapeDtypeStruct((2 * NPAD, H), jnp.float32),
    )(s_flat, s_flat, g_flat, g_flat, degp, b1, W2)


def _tc3_body(sa_ref, sb_ref, ga_ref, gb_ref, dp_ref, b_ref, wl_ref, bl_ref,
              bat_ref, o_ref):
    r = pl.program_id(0)
    dinv = _dinv(dp_ref)[:, None]
    ha = jnp.maximum((sa_ref[...] + ga_ref[...]) * dinv + b_ref[0, :H], 0.0)
    hb = jnp.maximum((sb_ref[...] + gb_ref[...]) * dinv + b_ref[0, H:], 0.0)
    y = (jnp.dot(ha, wl_ref[:H, :], preferred_element_type=jnp.float32)
         + jnp.dot(hb, wl_ref[H:, :], preferred_element_type=jnp.float32))
    gids = lax.broadcasted_iota(jnp.int32, (R, G), 1)
    p = (bat_ref[...] == gids).astype(jnp.float32)
    contrib = lax.dot_general(p, y, (((0,), (0,)), ((), ())),
                              preferred_element_type=jnp.float32)

    @pl.when(r == 0)
    def _():
        o_ref[...] = contrib + bl_ref[...]

    @pl.when(r != 0)
    def _():
        o_ref[...] = o_ref[...] + contrib


def _tc3_call(s_flat, g_flat, degp, b2, Wl, bl, batchp):
    half_a = pl.BlockSpec((R, H), lambda r: (r, 0))
    half_b = pl.BlockSpec((R, H), lambda r: (NBLK + r, 0))
    return pl.pallas_call(
        _tc3_body,
        grid=(NBLK,),
        in_specs=[
            half_a, half_b, half_a, half_b,
            pl.BlockSpec((R, 2), lambda r: (r, 0)),
            pl.BlockSpec((1, D), lambda r: (0, 0)),
            pl.BlockSpec((D, 64), lambda r: (0, 0)),
            pl.BlockSpec((1, 64), lambda r: (0, 0)),
            pl.BlockSpec((R, 1), lambda r: (r, 0)),
        ],
        out_specs=pl.BlockSpec((G, 64), lambda r: (0, 0)),
        out_shape=jax.ShapeDtypeStruct((G, 64), jnp.float32),
    )(s_flat, s_flat, g_flat, g_flat, degp, b2, Wl, bl, batchp)


def kernel(x, edge_index, batch, W1, b1, W2, b2, Wl, bl):
    src = edge_index[0]
    dst = edge_index[1]

    xp = jnp.pad(x, ((0, NPAD - N), (0, 0)))
    batchp = jnp.concatenate(
        [batch, jnp.full((NPAD - N,), G, dtype=batch.dtype)]).reshape(NPAD, 1)
    src_t = src.reshape(16, NCH // SUB, SUB, EC)
    src_slabs = jnp.stack([src_t, src_t + NPAD])
    dst_slabs = dst.reshape(16, NCH // SUB, SUB, EC)
    dst_deg = dst.reshape(32, 5, DCH // 5, DC)
    b1r = b1.reshape(1, D)
    b2r = b2.reshape(1, D)
    blr = bl.reshape(1, 64)

    degp = _deg_call(dst_deg)
    g1 = _tc1_call(xp, W1, degp)
    s1 = _prop_call(g1, src_slabs, dst_slabs)
    g2 = _tc2_call(s1, g1, degp, b1r, W2)
    s2 = _prop_call(g2, src_slabs, dst_slabs)
    return _tc3_call(s2, g2, degp, b2r, Wl, blr, batchp)


# speedup vs baseline: 20.2874x; 1.0685x over previous
import functools

import jax
import jax.numpy as jnp
from jax import lax
from jax.experimental import pallas as pl
from jax.experimental.pallas import tpu as pltpu
from jax.experimental.pallas import tpu_sc as plsc

N = 10000
NPAD = 10240
E = 160000
D = 256
H = 128
G = 128
R = 1024
NBLK = NPAD // R

EC = 80
NCH = E // (16 * EC)
SUB = 25
DC = 40
DCH = E // (32 * DC)
RPT = NPAD // 16

@functools.cache
def _mesh():
    return plsc.VectorSubcoreMesh(
        core_axis_name="c", subcore_axis_name="s", num_cores=2,
        num_subcores=16)


def _deg_body(dst_hbm, degp_hbm, dst_v, ones_v, buf_v, deg_sh):
    cid = lax.axis_index("c")
    sid = lax.axis_index("s")
    t = cid * 16 + sid

    for i in range(RPT // 16):
        buf_v[pl.ds(i * 16, 16)] = jnp.zeros((16,), jnp.float32)
    pltpu.sync_copy(buf_v, deg_sh.at[pl.ds(sid * RPT, RPT)])
    plsc.subcore_barrier()

    for i in range(DC // 16 + 1):
        ones_v[pl.ds(i * 16, 16)] = jnp.ones((16,), jnp.float32)

    def outer(o, _):
        pltpu.sync_copy(dst_hbm.at[t, o], dst_v)

        def chunk(j, _):
            pltpu.sync_copy(ones_v.at[pl.ds(0, DC)], deg_sh.at[dst_v.at[j]],
                            add=True)
            return 0
        lax.fori_loop(0, DCH // 5, chunk, 0)
        return 0
    lax.fori_loop(0, 5, outer, 0)
    plsc.subcore_barrier()

    pltpu.sync_copy(deg_sh.at[pl.ds(sid * RPT, RPT)], buf_v)
    pltpu.sync_copy(buf_v, degp_hbm.at[pl.ds(cid * NPAD + sid * RPT, RPT)])


def _deg_call(dst_slabs):
    flat = pl.kernel(
        _deg_body,
        out_type=jax.ShapeDtypeStruct((2 * NPAD,), jnp.float32),
        mesh=_mesh(),
        scratch_types=[
            pltpu.VMEM((DCH // 5, DC), jnp.int32),
            pltpu.VMEM((48,), jnp.float32),
            pltpu.VMEM((RPT,), jnp.float32),
            pltpu.VMEM_SHARED((NPAD,), jnp.float32),
        ],
    )(dst_slabs)
    return jnp.stack([flat[:NPAD], flat[NPAD:]], axis=1)


def _prop_body(g_hbm, src_hbm, dst_hbm, s_hbm,
               src_v, dst_v, rows0_v, rows1_v, rows2_v, rows3_v, accum,
               sem0, sem1, sem2, sem3):
    cid = lax.axis_index("c")
    sid = lax.axis_index("s")

    def zrow(r, _):
        for b in range(H // 16):
            rows0_v[r, pl.ds(b * 16, 16)] = jnp.zeros((16,), jnp.float32)
        return 0
    lax.fori_loop(0, EC, zrow, 0)
    for k in range(RPT // EC):
        pltpu.sync_copy(rows0_v, accum.at[pl.ds(sid * RPT + k * EC, EC), :])
    plsc.subcore_barrier()

    rows = (rows0_v, rows1_v, rows2_v, rows3_v)
    sems = (sem0, sem1, sem2, sem3)
    nb = len(rows)

    def outer(o, _):
        pltpu.sync_copy(src_hbm.at[cid, sid, o], src_v)
        pltpu.sync_copy(dst_hbm.at[sid, o], dst_v)
        for b in range(nb - 1):
            pltpu.async_copy(g_hbm.at[src_v.at[b]], rows[b], sems[b])

        def chunk(j, _):
            for b in range(nb):
                @pl.when(j % nb == b)
                def _():
                    pltpu.make_async_copy(
                        g_hbm.at[src_v.at[j]], rows[b], sems[b]).wait()

                    @pl.when(j < SUB - (nb - 1))
                    def _():
                        pltpu.async_copy(
                            g_hbm.at[src_v.at[j + nb - 1]], rows[(b - 1) % nb],
                            sems[(b - 1) % nb])
                    pltpu.sync_copy(rows[b], accum.at[dst_v.at[j]], add=True)
            return 0
        lax.fori_loop(0, SUB, chunk, 0)
        return 0
    lax.fori_loop(0, NCH // SUB, outer, 0)
    plsc.subcore_barrier()

    pltpu.sync_copy(accum.at[pl.ds(sid * RPT, RPT), :],
                    s_hbm.at[pl.ds(cid * NPAD + sid * RPT, RPT), :])


def _prop_call(g_flat, src_slabs, dst_slabs):
    return pl.kernel(
        _prop_body,
        out_type=jax.ShapeDtypeStruct((2 * NPAD, H), jnp.float32),
        mesh=_mesh(),
        scratch_types=[
            pltpu.VMEM((SUB, EC), jnp.int32),
            pltpu.VMEM((SUB, EC), jnp.int32),
            pltpu.VMEM((EC, H), jnp.float32),
            pltpu.VMEM((EC, H), jnp.float32),
            pltpu.VMEM((EC, H), jnp.float32),
            pltpu.VMEM((EC, H), jnp.float32),
            pltpu.VMEM_SHARED((NPAD, H), jnp.float32),
            pltpu.SemaphoreType.DMA,
            pltpu.SemaphoreType.DMA,
            pltpu.SemaphoreType.DMA,
            pltpu.SemaphoreType.DMA,
        ],
    )(g_flat, src_slabs, dst_slabs)


def _dinv(dp_ref):
    deg = dp_ref[:, 0] + dp_ref[:, 1] + 1.0
    return lax.rsqrt(deg)


def _tc1_body(x_ref, w_ref, dp_ref, o_ref):
    dinv = _dinv(dp_ref)[:, None]
    h = jnp.dot(x_ref[...], w_ref[...], preferred_element_type=jnp.float32)
    o_ref[0] = h[:, :H] * dinv
    o_ref[1] = h[:, H:] * dinv


def _tc1_call(xp, W1, degp):
    out = pl.pallas_call(
        _tc1_body,
        grid=(NBLK,),
        in_specs=[
            pl.BlockSpec((R, D), lambda r: (r, 0)),
            pl.BlockSpec((D, D), lambda r: (0, 0)),
            pl.BlockSpec((R, 2), lambda r: (r, 0)),
        ],
        out_specs=pl.BlockSpec((2, R, H), lambda r: (0, r, 0)),
        out_shape=jax.ShapeDtypeStruct((2, NPAD, H), jnp.float32),
    )(xp, W1, degp)
    return out.reshape(2 * NPAD, H)


def _tc2_body(sa_ref, sb_ref, ga_ref, gb_ref, dp_ref, b_ref, w_ref, o_ref):
    dinv = _dinv(dp_ref)[:, None]
    ha = jnp.maximum((sa_ref[...] + ga_ref[...]) * dinv + b_ref[0, :H], 0.0)
    hb = jnp.maximum((sb_ref[...] + gb_ref[...]) * dinv + b_ref[0, H:], 0.0)
    h2 = (jnp.dot(ha, w_ref[:H, :], preferred_element_type=jnp.float32)
          + jnp.dot(hb, w_ref[H:, :], preferred_element_type=jnp.float32))
    o_ref[0] = h2[:, :H] * dinv
    o_ref[1] = h2[:, H:] * dinv


def _tc2_call(s_flat, g_flat, degp, b1, W2):
    half_a = pl.BlockSpec((R, H), lambda r: (r, 0))
    half_b = pl.BlockSpec((R, H), lambda r: (NBLK + r, 0))
    out = pl.pallas_call(
        _tc2_body,
        grid=(NBLK,),
        in_specs=[
            half_a, half_b, half_a, half_b,
            pl.BlockSpec((R, 2), lambda r: (r, 0)),
            pl.BlockSpec((1, D), lambda r: (0, 0)),
            pl.BlockSpec((D, D), lambda r: (0, 0)),
        ],
        out_specs=pl.BlockSpec((2, R, H), lambda r: (0, r, 0)),
        out_shape=jax.ShapeDtypeStruct((2, NPAD, H), jnp.float32),
    )(s_flat, s_flat, g_flat, g_flat, degp, b1, W2)
    return out.reshape(2 * NPAD, H)


def _tc3_body(sa_ref, sb_ref, ga_ref, gb_ref, dp_ref, b_ref, wl_ref, bl_ref,
              bat_ref, o_ref):
    r = pl.program_id(0)
    dinv = _dinv(dp_ref)[:, None]
    ha = jnp.maximum((sa_ref[...] + ga_ref[...]) * dinv + b_ref[0, :H], 0.0)
    hb = jnp.maximum((sb_ref[...] + gb_ref[...]) * dinv + b_ref[0, H:], 0.0)
    y = (jnp.dot(ha, wl_ref[:H, :], preferred_element_type=jnp.float32)
         + jnp.dot(hb, wl_ref[H:, :], preferred_element_type=jnp.float32))
    gids = lax.broadcasted_iota(jnp.int32, (R, G), 1)
    p = (bat_ref[...] == gids).astype(jnp.float32)
    contrib = lax.dot_general(p, y, (((0,), (0,)), ((), ())),
                              preferred_element_type=jnp.float32)

    @pl.when(r == 0)
    def _():
        o_ref[...] = contrib + bl_ref[...]

    @pl.when(r != 0)
    def _():
        o_ref[...] = o_ref[...] + contrib


def _tc3_call(s_flat, g_flat, degp, b2, Wl, bl, batchp):
    half_a = pl.BlockSpec((R, H), lambda r: (r, 0))
    half_b = pl.BlockSpec((R, H), lambda r: (NBLK + r, 0))
    return pl.pallas_call(
        _tc3_body,
        grid=(NBLK,),
        in_specs=[
            half_a, half_b, half_a, half_b,
            pl.BlockSpec((R, 2), lambda r: (r, 0)),
            pl.BlockSpec((1, D), lambda r: (0, 0)),
            pl.BlockSpec((D, 64), lambda r: (0, 0)),
            pl.BlockSpec((1, 64), lambda r: (0, 0)),
            pl.BlockSpec((R, 1), lambda r: (r, 0)),
        ],
        out_specs=pl.BlockSpec((G, 64), lambda r: (0, 0)),
        out_shape=jax.ShapeDtypeStruct((G, 64), jnp.float32),
    )(s_flat, s_flat, g_flat, g_flat, degp, b2, Wl, bl, batchp)


def kernel(x, edge_index, batch, W1, b1, W2, b2, Wl, bl):
    src = edge_index[0]
    dst = edge_index[1]

    xp = jnp.pad(x, ((0, NPAD - N), (0, 0)))
    batchp = jnp.concatenate(
        [batch, jnp.full((NPAD - N,), G, dtype=batch.dtype)]).reshape(NPAD, 1)
    src_t = src.reshape(16, NCH // SUB, SUB, EC)
    src_slabs = jnp.stack([src_t, src_t + NPAD])
    dst_slabs = dst.reshape(16, NCH // SUB, SUB, EC)
    dst_deg = dst.reshape(32, 5, DCH // 5, DC)
    b1r = b1.reshape(1, D)
    b2r = b2.reshape(1, D)
    blr = bl.reshape(1, 64)

    degp = _deg_call(dst_deg)
    g1 = _tc1_call(xp, W1, degp)
    s1 = _prop_call(g1, src_slabs, dst_slabs)
    g2 = _tc2_call(s1, g1, degp, b1r, W2)
    s2 = _prop_call(g2, src_slabs, dst_slabs)
    return _tc3_call(s2, g2, degp, b2r, Wl, blr, batchp)
